# baseline (speedup 1.0000x reference)
import functools

import jax
import jax.numpy as jnp
import numpy as np
from jax import lax
from jax.experimental import pallas as pl
from jax.experimental.pallas import tpu as pltpu
from jax.experimental.pallas import tpu_sc as plsc

N_NODES = 10000
D = 129
D_PAD = 144
N_EDGES = 320000

NC = 2
NS = 16
NW = NC * NS
E_PER_W = N_EDGES // NW
CHUNK = 80
N_CHUNKS = E_PER_W // CHUNK
GROUPS = CHUNK // 16

_LN2 = 0.6931471805599453


def _rsqrt(z):
    i = lax.bitcast_convert_type(z, jnp.int32)
    i = jnp.int32(0x5F3759DF) - (i >> 1)
    y = lax.bitcast_convert_type(i, jnp.float32)
    for _ in range(3):
        y = y * (1.5 - 0.5 * z * y * y)
    return y


def _log(w):
    i = lax.bitcast_convert_type(w, jnp.int32)
    e = (i >> 23) - 127
    m = lax.bitcast_convert_type(
        (i & jnp.int32(0x007FFFFF)) | jnp.int32(0x3F800000), jnp.float32)
    adj = m > 1.4142135
    m = jnp.where(adj, m * 0.5, m)
    e = jnp.where(adj, e + 1, e)
    s = (m - 1.0) / (m + 1.0)
    s2 = s * s
    ll = 2.0 * s * (1.0 + s2 * (1 / 3 + s2 * (1 / 5 + s2 * (1 / 7 + s2 * (1 / 9)))))
    return e.astype(jnp.float32) * jnp.float32(_LN2) + ll


def _edge_math(mdot):
    theta = jnp.maximum(-mdot, 1.0 + 1e-6)
    zz = (theta - 1.0) * (theta + 1.0)
    w = theta + zz * _rsqrt(zz)
    lw = _log(w)
    sq = lw * lw
    return 1.0 / (1.0 + jnp.exp(sq - 2.0))


_GDN = lax.GatherDimensionNumbers(
    offset_dims=(), collapsed_slice_dims=(0,), start_index_map=(0,))


def _lanes():
    return jnp.arange(16, dtype=jnp.int32)


def _lane_perm(v, s):
    idx = _lanes() ^ s
    return lax.gather(v, idx[:, None], _GDN, slice_sizes=(1,),
                      mode=lax.GatherScatterMode.PROMISE_IN_BOUNDS)


def _merge(u, v, s):
    sel = (_lanes() & s) == 0
    return jnp.where(sel, u + _lane_perm(u, s), v + _lane_perm(v, s))


def _compute_chunk(g, gbuf, out_v):
    def group_body(g2, carry):
        base = g2 * 32

        def edge_acc(e):
            rx = base + 2 * e
            x = gbuf[rx, pl.ds(0, 16)]
            y = gbuf[rx + 1, pl.ds(0, 16)]
            acc = x * y
            for k in range(1, 8):
                x = gbuf[rx, pl.ds(16 * k, 16)]
                y = gbuf[rx + 1, pl.ds(16 * k, 16)]
                acc = acc + x * y
            return acc

        quads = []
        for b in range(4):
            a0, a1, a2, a3 = (edge_acc(4 * b + j) for j in range(4))
            quads.append(_merge(_merge(a0, a1, 1), _merge(a2, a3, 1), 2))
        msum = _merge(_merge(quads[0], quads[1], 4),
                      _merge(quads[2], quads[3], 4), 8)
        rows_x = _lanes() * 2 + base
        x0 = plsc.load_gather(gbuf, [rows_x, jnp.zeros((16,), jnp.int32)])
        y0 = plsc.load_gather(gbuf, [rows_x + 1, jnp.zeros((16,), jnp.int32)])
        c128 = jnp.full((16,), 128, jnp.int32)
        x128 = plsc.load_gather(gbuf, [rows_x, c128])
        y128 = plsc.load_gather(gbuf, [rows_x + 1, c128])
        mdot = msum + x128 * y128 - 2.0 * (x0 * y0)
        p = _edge_math(mdot)
        out_v[pl.ds(g * CHUNK + g2 * 16, 16)] = p
        return carry

    lax.fori_loop(0, GROUPS, group_body, 0)


def _start_gathers(h_src, idx_v, g, gbuf, sem_x, sem_y):
    cx = pltpu.async_copy(h_src.at[idx_v.at[g, 0]], gbuf.at[pl.ds(0, 80)], sem_x)
    cy = pltpu.async_copy(h_src.at[idx_v.at[g, 1]], gbuf.at[pl.ds(80, 80)], sem_y)
    return cx, cy


def _wait_gathers(h_src, idx_v, g, gbuf, sem_x, sem_y):
    pltpu.make_async_copy(h_src.at[idx_v.at[g, 0]], gbuf.at[pl.ds(0, 80)], sem_x).wait()
    pltpu.make_async_copy(h_src.at[idx_v.at[g, 1]], gbuf.at[pl.ds(80, 80)], sem_y).wait()


def _sc_body(h_hbm, idx_hbm, out_hbm, idx_v, g0, g1, out_v,
             sx0, sy0, sx1, sy1):
    wid = lax.axis_index("s") * NC + lax.axis_index("c")

    pltpu.sync_copy(idx_hbm.at[pl.ds(wid * N_CHUNKS, N_CHUNKS)], idx_v)

    _start_gathers(h_hbm, idx_v, 0, g0, sx0, sy0)

    def pair_body(i, carry):
        ca = 2 * i
        cb = 2 * i + 1
        _start_gathers(h_hbm, idx_v, cb, g1, sx1, sy1)
        _wait_gathers(h_hbm, idx_v, ca, g0, sx0, sy0)
        _compute_chunk(ca, g0, out_v)
        _start_gathers(h_hbm, idx_v, cb + 1, g0, sx0, sy0)
        _wait_gathers(h_hbm, idx_v, cb, g1, sx1, sy1)
        _compute_chunk(cb, g1, out_v)
        return carry

    lax.fori_loop(0, (N_CHUNKS - 1) // 2, pair_body, 0)

    last = N_CHUNKS - 1
    _wait_gathers(h_hbm, idx_v, last, g0, sx0, sy0)
    _compute_chunk(last, g0, out_v)

    pltpu.sync_copy(out_v, out_hbm.at[pl.ds(wid * E_PER_W, E_PER_W)])


@jax.jit
def _lp_decode(h, idx3):
    mesh = plsc.VectorSubcoreMesh(core_axis_name="c", subcore_axis_name="s")
    run = pl.kernel(
        _sc_body,
        out_type=jax.ShapeDtypeStruct((N_EDGES,), jnp.float32),
        mesh=mesh,
        compiler_params=pltpu.CompilerParams(
            use_tc_tiling_on_sc=False, needs_layout_passes=False),
        scratch_types=[
            pltpu.VMEM((N_CHUNKS, 2, CHUNK), jnp.int32),
            pltpu.VMEM((2 * CHUNK, D_PAD), jnp.float32),
            pltpu.VMEM((2 * CHUNK, D_PAD), jnp.float32),
            pltpu.VMEM((E_PER_W,), jnp.float32),
            pltpu.SemaphoreType.DMA,
            pltpu.SemaphoreType.DMA,
            pltpu.SemaphoreType.DMA,
            pltpu.SemaphoreType.DMA,
        ],
    )
    return run(h, idx3)


def kernel(h, idx):
    idx3 = idx.reshape(N_EDGES // CHUNK, 2, CHUNK)
    hp = jnp.pad(h, ((0, 0), (0, D_PAD - D)))
    return _lp_decode(hp, idx3)


# speedup vs baseline: 1.2565x; 1.2565x over previous
import functools

import jax
import jax.numpy as jnp
from jax import lax
from jax.experimental import pallas as pl
from jax.experimental.pallas import tpu as pltpu
from jax.experimental.pallas import tpu_sc as plsc

N_NODES = 10000
D = 129
D_PAD = 144
N_EDGES = 320000

NC = 2
NS = 16
NW = NC * NS
E_PER_W = N_EDGES // NW
CHUNK = 80
N_CHUNKS = E_PER_W // CHUNK
GROUPS = CHUNK // 16
NODES_PER_TILE = N_NODES // NS

_LN2 = 0.6931471805599453


def _rsqrt(z):
    i = lax.bitcast_convert_type(z, jnp.int32)
    i = jnp.int32(0x5F3759DF) - (i >> 1)
    y = lax.bitcast_convert_type(i, jnp.float32)
    for _ in range(3):
        y = y * (1.5 - 0.5 * z * y * y)
    return y


def _log(w):
    i = lax.bitcast_convert_type(w, jnp.int32)
    e = (i >> 23) - 127
    m = lax.bitcast_convert_type(
        (i & jnp.int32(0x007FFFFF)) | jnp.int32(0x3F800000), jnp.float32)
    adj = m > 1.4142135
    m = jnp.where(adj, m * 0.5, m)
    e = jnp.where(adj, e + 1, e)
    s = (m - 1.0) / (m + 1.0)
    s2 = s * s
    ll = 2.0 * s * (1.0 + s2 * (1 / 3 + s2 * (1 / 5 + s2 * (1 / 7 + s2 * (1 / 9)))))
    return e.astype(jnp.float32) * jnp.float32(_LN2) + ll


def _edge_math(mdot):
    theta = jnp.maximum(-mdot, 1.0 + 1e-6)
    zz = (theta - 1.0) * (theta + 1.0)
    w = theta + zz * _rsqrt(zz)
    lw = _log(w)
    sq = lw * lw
    return 1.0 / (1.0 + jnp.exp(sq - 2.0))


_GDN = lax.GatherDimensionNumbers(
    offset_dims=(), collapsed_slice_dims=(0,), start_index_map=(0,))


def _lanes():
    return jnp.arange(16, dtype=jnp.int32)


def _lane_perm(v, s):
    idx = _lanes() ^ s
    return lax.gather(v, idx[:, None], _GDN, slice_sizes=(1,),
                      mode=lax.GatherScatterMode.PROMISE_IN_BOUNDS)


def _merge(u, v, s):
    sel = (_lanes() & s) == 0
    return jnp.where(sel, u + _lane_perm(u, s), v + _lane_perm(v, s))


def _lorentz_sq_16rows(buf, base):
    def row_acc(j):
        r = base + j
        v = buf[r, pl.ds(0, 16)]
        acc = v * v
        for k in range(1, 8):
            v = buf[r, pl.ds(16 * k, 16)]
            acc = acc + v * v
        return acc

    quads = []
    for b in range(4):
        a0, a1, a2, a3 = (row_acc(4 * b + j) for j in range(4))
        quads.append(_merge(_merge(a0, a1, 1), _merge(a2, a3, 1), 2))
    ssum = _merge(_merge(quads[0], quads[1], 4),
                  _merge(quads[2], quads[3], 4), 8)
    rows = _lanes() + base
    v0 = plsc.load_gather(buf, [rows, jnp.zeros((16,), jnp.int32)])
    v128 = plsc.load_gather(buf, [rows, jnp.full((16,), 128, jnp.int32)])
    return ssum + v128 * v128 - 2.0 * (v0 * v0)


def _compute_chunk(g, gbuf, idx_v, nrm_v, out_v):
    def group_body(g2, carry):
        base = g2 * 16
        sjs = _lorentz_sq_16rows(gbuf, base)
        lv = _lanes() + base
        gsp = jnp.full((16,), g, jnp.int32)
        id0 = plsc.load_gather(idx_v, [gsp, jnp.zeros((16,), jnp.int32), lv])
        id1 = plsc.load_gather(idx_v, [gsp, jnp.ones((16,), jnp.int32), lv])
        nx = plsc.load_gather(nrm_v, [id0])
        ny = plsc.load_gather(nrm_v, [id1])
        mdot = (sjs - nx - ny) * 0.5
        p = _edge_math(mdot)
        out_v[pl.ds(g * CHUNK + g2 * 16, 16)] = p
        return carry

    lax.fori_loop(0, GROUPS, group_body, 0)


def _start_a(h_hbm, idx_v, g, gbuf, sem):
    gc = jnp.minimum(g, N_CHUNKS - 1)
    pltpu.async_copy(h_hbm.at[idx_v.at[gc, 0]], gbuf, sem)


def _wait_a(h_hbm, idx_v, g, gbuf, sem):
    gc = jnp.minimum(g, N_CHUNKS - 1)
    pltpu.make_async_copy(h_hbm.at[idx_v.at[gc, 0]], gbuf, sem).wait()


def _start_b(h_hbm, idx_v, g, gbuf, sem):
    pltpu.async_copy(h_hbm.at[idx_v.at[g, 1]], gbuf, sem, add=True)


def _wait_b(h_hbm, idx_v, g, gbuf, sem):
    pltpu.make_async_copy(h_hbm.at[idx_v.at[g, 1]], gbuf, sem).wait()


def _compute_norms(h_hbm, sid, stage, nstage, nrm_sp, nrm_v):
    for b in range(4):
        rbase = jnp.minimum((sid * 4 + b) * 160, N_NODES - 160)
        pltpu.sync_copy(h_hbm.at[pl.ds(rbase, 160)], stage)

        def nb(j, carry):
            nstage[pl.ds(j * 16, 16)] = _lorentz_sq_16rows(stage, j * 16)
            return carry

        lax.fori_loop(0, 10, nb, 0)
        pltpu.sync_copy(nstage, nrm_sp.at[pl.ds(rbase, 160)])
    plsc.subcore_barrier()
    pltpu.sync_copy(nrm_sp, nrm_v)


def _sc_body(h_hbm, idx_hbm, out_hbm, idx_v, g0, g1, out_v,
             stage, nstage, nrm_v, nrm_sp,
             sa0, sb0, sa1, sb1):
    cid = lax.axis_index("c")
    sid = lax.axis_index("s")
    wid = sid * NC + cid

    _compute_norms(h_hbm, sid, stage, nstage, nrm_sp, nrm_v)

    pltpu.sync_copy(idx_hbm.at[pl.ds(wid * N_CHUNKS, N_CHUNKS)], idx_v)

    _start_a(h_hbm, idx_v, 0, g0, sa0)
    _start_a(h_hbm, idx_v, 1, g1, sa1)

    def pair_body(i, carry):
        ca = 2 * i
        cb = 2 * i + 1
        _wait_a(h_hbm, idx_v, ca, g0, sa0)
        _start_b(h_hbm, idx_v, ca, g0, sb0)
        _wait_b(h_hbm, idx_v, ca, g0, sb0)
        _compute_chunk(ca, g0, idx_v, nrm_v, out_v)
        _start_a(h_hbm, idx_v, ca + 2, g0, sa0)
        _wait_a(h_hbm, idx_v, cb, g1, sa1)
        _start_b(h_hbm, idx_v, cb, g1, sb1)
        _wait_b(h_hbm, idx_v, cb, g1, sb1)
        _compute_chunk(cb, g1, idx_v, nrm_v, out_v)
        _start_a(h_hbm, idx_v, cb + 2, g1, sa1)
        return carry

    lax.fori_loop(0, (N_CHUNKS - 1) // 2, pair_body, 0)

    last = N_CHUNKS - 1
    _wait_a(h_hbm, idx_v, last, g1, sa1)
    _wait_a(h_hbm, idx_v, last, g0, sa0)
    _start_b(h_hbm, idx_v, last, g0, sb0)
    _wait_b(h_hbm, idx_v, last, g0, sb0)
    _compute_chunk(last, g0, idx_v, nrm_v, out_v)

    pltpu.sync_copy(out_v, out_hbm.at[pl.ds(wid * E_PER_W, E_PER_W)])


@jax.jit
def _lp_decode(h, idx3):
    mesh = plsc.VectorSubcoreMesh(core_axis_name="c", subcore_axis_name="s")
    run = pl.kernel(
        _sc_body,
        out_type=jax.ShapeDtypeStruct((N_EDGES,), jnp.float32),
        mesh=mesh,
        compiler_params=pltpu.CompilerParams(
            use_tc_tiling_on_sc=False, needs_layout_passes=False),
        scratch_types=[
            pltpu.VMEM((N_CHUNKS, 2, CHUNK), jnp.int32),
            pltpu.VMEM((CHUNK, D_PAD), jnp.float32),
            pltpu.VMEM((CHUNK, D_PAD), jnp.float32),
            pltpu.VMEM((E_PER_W,), jnp.float32),
            pltpu.VMEM((160, D_PAD), jnp.float32),
            pltpu.VMEM((160,), jnp.float32),
            pltpu.VMEM((N_NODES,), jnp.float32),
            pltpu.VMEM_SHARED((N_NODES,), jnp.float32),
            pltpu.SemaphoreType.DMA,
            pltpu.SemaphoreType.DMA,
            pltpu.SemaphoreType.DMA,
            pltpu.SemaphoreType.DMA,
        ],
    )
    return run(h, idx3)


def kernel(h, idx):
    idx3 = jnp.transpose(idx.reshape(N_EDGES // CHUNK, CHUNK, 2), (0, 2, 1))
    hp = jnp.pad(h, ((0, 0), (0, D_PAD - D)))
    return _lp_decode(hp, idx3)
